# Initial kernel scaffold; baseline (speedup 1.0000x reference)
import jax
import jax.numpy as jnp
from jax.experimental import pallas as pl


def kernel(adj_matrix_batch, edges_matrix_batch, features_batch):
    raise NotImplementedError("write your pallas kernel here")


# speedup vs baseline: 29.9514x; 29.9514x over previous
import functools

import jax
import jax.numpy as jnp
from jax import lax
from jax.experimental import pallas as pl
from jax.experimental.pallas import tpu as pltpu
from jax.experimental.pallas import tpu_sc as plsc

B, N, D_EDGE, D_FEAT = 8, 512, 4, 128
K = 17
KP = 32
NU = B * N
NPAIR = K * K
PAIR_PAD = 304
NCH = PAIR_PAD // 16
NMASK = (NPAIR * D_EDGE + 15) // 16
UNSEEN = 1 << 14
POPPED = 1 << 15

NC, NS = 2, 16
NW = NC * NS
UPW = NU // NW


def _bfs_body(adj_ref, out_ref):
    A = adj_ref[0]
    Ab = (A != 0.0).astype(jnp.bfloat16)
    iu = lax.broadcasted_iota(jnp.int32, (N, N), 1)
    iv = lax.broadcasted_iota(jnp.int32, (N, N), 0)
    disc = jnp.where(iu == iv, 0, UNSEEN)
    colk = lax.broadcasted_iota(jnp.int32, (N, KP), 1)
    acc = jnp.zeros((N, KP), jnp.int32)
    for k in range(K):
        key = disc * N + iu
        m = jnp.min(key, axis=1, keepdims=True)
        p = jnp.bitwise_and(m, N - 1)
        acc = jnp.where(colk == k, p, acc)
        disc = jnp.where(iu == p, POPPED, disc)
        oh = (iu == p).astype(jnp.bfloat16)
        row = jnp.dot(oh, Ab, preferred_element_type=jnp.float32)
        disc = jnp.where((row > 0.5) & (disc == UNSEEN), k + 1, disc)
    b = pl.program_id(0)
    out_ref[0] = acc + b * N


def _bfs(adj):
    return pl.pallas_call(
        _bfs_body,
        grid=(B,),
        in_specs=[pl.BlockSpec((1, N, N), lambda b: (b, 0, 0))],
        out_specs=pl.BlockSpec((1, N, KP), lambda b: (b, 0, 0)),
        out_shape=jax.ShapeDtypeStruct((B, N, KP), jnp.int32),
    )(adj)


@functools.partial(
    pl.kernel,
    mesh=plsc.VectorSubcoreMesh(core_axis_name="c", subcore_axis_name="s"),
    compiler_params=pltpu.CompilerParams(
        needs_layout_passes=False, use_tc_tiling_on_sc=False),
    out_type=[
        jax.ShapeDtypeStruct((NU, K, D_FEAT), jnp.float32),
        jax.ShapeDtypeStruct((NU, PAIR_PAD), jnp.float32),
        jax.ShapeDtypeStruct((NU, PAIR_PAD, D_EDGE), jnp.float32),
    ],
    scratch_types=[
        pltpu.VMEM((KP,), jnp.int32),
        pltpu.VMEM((PAIR_PAD,), jnp.int32),
        pltpu.VMEM((PAIR_PAD,), jnp.int32),
        pltpu.VMEM((PAIR_PAD,), jnp.int32),
        pltpu.VMEM((K, D_FEAT), jnp.float32),
        pltpu.VMEM((PAIR_PAD, 16), jnp.float32),
        pltpu.VMEM((PAIR_PAD, 16), jnp.float32),
        pltpu.VMEM((PAIR_PAD,), jnp.float32),
        pltpu.VMEM((PAIR_PAD, D_EDGE), jnp.float32),
        pltpu.VMEM((PAIR_PAD,), jnp.int32),
        pltpu.VMEM((PAIR_PAD,), jnp.int32),
        pltpu.VMEM((PAIR_PAD,), jnp.int32),
        pltpu.VMEM((NMASK * 16,), jnp.int32),
        pltpu.VMEM((NMASK * 16,), jnp.int32),
        pltpu.SemaphoreType.DMA,
        pltpu.SemaphoreType.DMA,
        pltpu.SemaphoreType.DMA,
    ],
)
def _sc_gather(gidx_hbm, feats_hbm, adj16_hbm, edges16_hbm,
               wfeat_hbm, wadj_hbm, wedge_hbm,
               idxv, pairv, prav, prev, fbuf, abuf, ebuf, wadjv, webuf,
               ivt, jvt, ttab, mrt, mdt,
               sem_f, sem_a, sem_e):
    wid = lax.axis_index("s") * NC + lax.axis_index("c")
    lanes = lax.broadcasted_iota(jnp.int32, (16,), 0)

    def _bc(x):
        return jnp.broadcast_to(x, (16,))

    n16 = _bc(jnp.int32(N))
    k16 = _bc(jnp.int32(K))
    p16 = _bc(jnp.int32(NPAIR))
    z16 = _bc(jnp.int32(0))
    c3 = _bc(jnp.int32(3))
    c4s = _bc(jnp.int32(2))
    c7 = _bc(jnp.int32(7))
    c15 = _bc(jnp.int32(15))
    c511 = _bc(jnp.int32(N - 1))
    zf16 = _bc(jnp.float32(0.0))

    for c in range(NCH):
        tv = lanes + _bc(jnp.int32(c * 16))
        ok = tv < p16
        s = pl.ds(c * 16, 16)
        ttab[s] = tv
        ivt[s] = jnp.where(ok, tv // k16, z16)
        jvt[s] = jnp.where(ok, tv % k16, z16)
    for c in range(NMASK):
        tm = lanes + _bc(jnp.int32(c * 16))
        s = pl.ds(c * 16, 16)
        mrt[s] = lax.shift_right_logical(tm, c4s)
        mdt[s] = tm & c3

    def unit(t, carry):
        u = wid * UPW + t
        pltpu.sync_copy(gidx_hbm.at[u], idxv)
        cp_f = pltpu.async_copy(feats_hbm.at[idxv.at[pl.ds(0, K)]], fbuf, sem_f)
        for c in range(NCH):
            s = pl.ds(c * 16, 16)
            vi = plsc.load_gather(idxv, [ivt[s]])
            vj = plsc.load_gather(idxv, [jvt[s]])
            pair = vi * n16 + (vj & c511)
            pairv[s] = pair
            prav[s] = lax.shift_right_logical(pair, c4s + c4s)
            prev[s] = lax.shift_right_logical(pair, c4s)
        handles = [
            pltpu.async_copy(adj16_hbm.at[prav.at[pl.ds(0, 128)]],
                             abuf.at[pl.ds(0, 128)], sem_a),
            pltpu.async_copy(adj16_hbm.at[prav.at[pl.ds(128, 128)]],
                             abuf.at[pl.ds(128, 128)], sem_a),
            pltpu.async_copy(adj16_hbm.at[prav.at[pl.ds(256, 48)]],
                             abuf.at[pl.ds(256, 48)], sem_a),
            pltpu.async_copy(edges16_hbm.at[prev.at[pl.ds(0, 128)]],
                             ebuf.at[pl.ds(0, 128)], sem_e),
            pltpu.async_copy(edges16_hbm.at[prev.at[pl.ds(128, 128)]],
                             ebuf.at[pl.ds(128, 128)], sem_e),
            pltpu.async_copy(edges16_hbm.at[prev.at[pl.ds(256, 48)]],
                             ebuf.at[pl.ds(256, 48)], sem_e),
        ]
        for cp in handles:
            cp.wait()
        for c in range(NCH):
            s = pl.ds(c * 16, 16)
            rv = ttab[s]
            pv = plsc.load_gather(pairv, [rv])
            val = plsc.load_gather(abuf, [rv, pv & c15])
            wadjv[s] = val
        for c in range(NMASK):
            s = pl.ds(c * 16, 16)
            rv = mrt[s]
            dv = mdt[s]
            pv = plsc.load_gather(pairv, [rv])
            col = lax.shift_left(pv & c3, c4s) + dv
            ev = plsc.load_gather(ebuf, [rv, col])
            av = plsc.load_gather(wadjv, [rv])
            plsc.store_scatter(webuf, [rv, dv],
                               jnp.where(av != zf16, ev, zf16))
        cp_f.wait()
        pltpu.sync_copy(fbuf, wfeat_hbm.at[u])
        pltpu.sync_copy(wadjv, wadj_hbm.at[u])
        pltpu.sync_copy(webuf, wedge_hbm.at[u])
        return carry

    lax.fori_loop(0, UPW, unit, None)


def kernel(adj_matrix_batch, edges_matrix_batch, features_batch):
    gidx = _bfs(adj_matrix_batch).reshape(NU, KP)
    feats_flat = features_batch.reshape(NU, D_FEAT)
    adj16 = adj_matrix_batch.reshape(B * N * N // 16, 16)
    edges16 = edges_matrix_batch.reshape(B * N * N // 4, 16)
    wfeat, wadj, wedge = _sc_gather(gidx, feats_flat, adj16, edges16)
    w_feat = wfeat.reshape(B, N, K, D_FEAT)
    w_adj = wadj[:, :NPAIR].reshape(B, N, K, K)
    w_edges = wedge[:, :NPAIR, :].reshape(B, N, K, K, D_EDGE)
    return (w_adj, w_edges, w_feat)


# speedup vs baseline: 173.6310x; 5.7971x over previous
import functools

import jax
import jax.numpy as jnp
from jax import lax
from jax.experimental import pallas as pl
from jax.experimental.pallas import tpu as pltpu
from jax.experimental.pallas import tpu_sc as plsc

B, N, D_EDGE, D_FEAT = 8, 512, 4, 128
K = 17
KP = 32
NU = B * N
NPAIR = K * K
PAIR_PAD = 304
NCH = PAIR_PAD // 16
NED = NPAIR * D_EDGE
NEDP = 1168
NMASK = NEDP // 16
WEPAD = PAIR_PAD * D_EDGE
UNSEEN = 1 << 14
POPPED = 1 << 15

NC, NS = 2, 16
NW = NC * NS
UPW = NU // NW


def _bfs_body(adj_ref, out_ref):
    A = adj_ref[0]
    Ab = (A != 0.0).astype(jnp.bfloat16)
    iu = lax.broadcasted_iota(jnp.int32, (N, N), 1)
    iv = lax.broadcasted_iota(jnp.int32, (N, N), 0)
    disc = jnp.where(iu == iv, 0, UNSEEN)
    colk = lax.broadcasted_iota(jnp.int32, (N, KP), 1)
    acc = jnp.zeros((N, KP), jnp.int32)
    for k in range(K):
        key = disc * N + iu
        m = jnp.min(key, axis=1, keepdims=True)
        p = jnp.bitwise_and(m, N - 1)
        acc = jnp.where(colk == k, p, acc)
        disc = jnp.where(iu == p, POPPED, disc)
        oh = (iu == p).astype(jnp.bfloat16)
        row = jnp.dot(oh, Ab, preferred_element_type=jnp.float32)
        disc = jnp.where((row > 0.5) & (disc == UNSEEN), k + 1, disc)
    b = pl.program_id(0)
    out_ref[0] = acc + b * N


def _bfs(adj):
    return pl.pallas_call(
        _bfs_body,
        grid=(B,),
        in_specs=[pl.BlockSpec((1, N, N), lambda b: (b, 0, 0))],
        out_specs=pl.BlockSpec((1, N, KP), lambda b: (b, 0, 0)),
        out_shape=jax.ShapeDtypeStruct((B, N, KP), jnp.int32),
    )(adj)


@functools.partial(
    pl.kernel,
    mesh=plsc.VectorSubcoreMesh(core_axis_name="c", subcore_axis_name="s"),
    compiler_params=pltpu.CompilerParams(
        needs_layout_passes=False, use_tc_tiling_on_sc=False),
    out_type=[
        jax.ShapeDtypeStruct((B * K * N, D_FEAT), jnp.float32),
        jax.ShapeDtypeStruct((NU, PAIR_PAD), jnp.float32),
        jax.ShapeDtypeStruct((NU, WEPAD), jnp.float32),
    ],
    scratch_types=[
        pltpu.VMEM((KP,), jnp.int32),
        pltpu.VMEM((PAIR_PAD,), jnp.int32),
        pltpu.VMEM((PAIR_PAD,), jnp.int32),
        pltpu.VMEM((PAIR_PAD,), jnp.int32),
        pltpu.VMEM((NEDP,), jnp.int32),
        pltpu.VMEM((NEDP,), jnp.int32),
        pltpu.VMEM((K, D_FEAT), jnp.float32),
        pltpu.VMEM((PAIR_PAD, 16), jnp.float32),
        pltpu.VMEM((NEDP, 16), jnp.float32),
        pltpu.VMEM((PAIR_PAD,), jnp.float32),
        pltpu.VMEM((WEPAD,), jnp.float32),
        pltpu.VMEM((PAIR_PAD,), jnp.int32),
        pltpu.VMEM((PAIR_PAD,), jnp.int32),
        pltpu.VMEM((PAIR_PAD,), jnp.int32),
        pltpu.VMEM((NEDP,), jnp.int32),
        pltpu.VMEM((NEDP,), jnp.int32),
        pltpu.VMEM((NEDP,), jnp.int32),
        pltpu.SemaphoreType.DMA,
        pltpu.SemaphoreType.DMA,
        pltpu.SemaphoreType.DMA,
        pltpu.SemaphoreType.DMA,
    ],
)
def _sc_gather(gidx_hbm, feats_hbm, adj16_hbm, edges16_hbm,
               wfeat_hbm, wadj_hbm, wedge_hbm,
               idxv, prav, acol, ebase, erow, ecol, fbuf, abuf, ebuf,
               wadjv, webuf, ivt, jvt, ttab, mtt, mrt, mdt,
               sem_f, sem_a, sem_e, sem_o):
    wid = lax.axis_index("s") * NC + lax.axis_index("c")
    lanes = lax.broadcasted_iota(jnp.int32, (16,), 0)

    def _bc(x):
        return jnp.broadcast_to(x, (16,))

    n16 = _bc(jnp.int32(N))
    k16 = _bc(jnp.int32(K))
    p16 = _bc(jnp.int32(NPAIR))
    z16 = _bc(jnp.int32(0))
    c3 = _bc(jnp.int32(3))
    c4 = _bc(jnp.int32(4))
    c7 = _bc(jnp.int32(7))
    c9 = _bc(jnp.int32(9))
    c15 = _bc(jnp.int32(15))
    c127 = _bc(jnp.int32(127))
    c511 = _bc(jnp.int32(N - 1))
    c2048 = _bc(jnp.int32(N * D_EDGE))
    zf16 = _bc(jnp.float32(0.0))

    for c in range(NCH):
        tv = lanes + _bc(jnp.int32(c * 16))
        ok = tv < p16
        s = pl.ds(c * 16, 16)
        ttab[s] = tv
        ivt[s] = jnp.where(ok, tv // k16, z16)
        jvt[s] = jnp.where(ok, tv % k16, z16)
    for c in range(NMASK):
        tm = lanes + _bc(jnp.int32(c * 16))
        s = pl.ds(c * 16, 16)
        mtt[s] = tm
        mrt[s] = lax.shift_right_logical(tm, c4s := _bc(jnp.int32(2)))
        mdt[s] = tm & c3

    def unit(t, carry):
        u = wid * UPW + t
        b = u // N
        v = u - b * N
        pltpu.sync_copy(gidx_hbm.at[u], idxv)
        cp_f = pltpu.async_copy(feats_hbm.at[idxv.at[pl.ds(0, K)]], fbuf, sem_f)
        for c in range(NCH):
            s = pl.ds(c * 16, 16)
            vi = plsc.load_gather(idxv, [ivt[s]])
            vj = plsc.load_gather(idxv, [jvt[s]])
            lj = vj & c511
            pair = vi * n16 + lj
            prav[s] = lax.shift_right_logical(pair, c4)
            acol[s] = pair & c15
            ebase[s] = (vi * c2048
                        + lax.shift_left(lax.shift_right_logical(lj, c7), c9)
                        + (lj & c127))
        for c in range(NMASK):
            s = pl.ds(c * 16, 16)
            base = plsc.load_gather(ebase, [mrt[s]])
            word = base + lax.shift_left(mdt[s], c7)
            erow[s] = lax.shift_right_logical(word, c4)
            ecol[s] = word & c15
        ha = [
            pltpu.async_copy(adj16_hbm.at[prav.at[pl.ds(0, 128)]],
                             abuf.at[pl.ds(0, 128)], sem_a),
            pltpu.async_copy(adj16_hbm.at[prav.at[pl.ds(128, 128)]],
                             abuf.at[pl.ds(128, 128)], sem_a),
            pltpu.async_copy(adj16_hbm.at[prav.at[pl.ds(256, 48)]],
                             abuf.at[pl.ds(256, 48)], sem_a),
        ]
        he = []
        for h in range(9):
            s = pl.ds(h * 128, 128)
            he.append(pltpu.async_copy(edges16_hbm.at[erow.at[s]],
                                       ebuf.at[s], sem_e))
        he.append(pltpu.async_copy(edges16_hbm.at[erow.at[pl.ds(1152, 16)]],
                                   ebuf.at[pl.ds(1152, 16)], sem_e))
        cp_f.wait()
        hf = []
        for k in range(K):
            hf.append(pltpu.async_copy(
                fbuf.at[k], wfeat_hbm.at[(b * K + k) * N + v], sem_o))
        for h in ha:
            h.wait()
        for c in range(NCH):
            s = pl.ds(c * 16, 16)
            val = plsc.load_gather(abuf, [ttab[s], acol[s]])
            wadjv[s] = val
        for h in he:
            h.wait()
        for c in range(NMASK):
            s = pl.ds(c * 16, 16)
            ev = plsc.load_gather(ebuf, [mtt[s], ecol[s]])
            av = plsc.load_gather(wadjv, [mrt[s]])
            webuf[s] = jnp.where(av != zf16, ev, zf16)
        pltpu.sync_copy(wadjv, wadj_hbm.at[u])
        pltpu.sync_copy(webuf, wedge_hbm.at[u])
        for h in hf:
            h.wait()
        return carry

    lax.fori_loop(0, UPW, unit, None)


def kernel(adj_matrix_batch, edges_matrix_batch, features_batch):
    gidx = _bfs(adj_matrix_batch).reshape(NU, KP)
    feats_flat = features_batch.reshape(NU, D_FEAT)
    adj16 = adj_matrix_batch.reshape(B * N * N // 16, 16)
    edges16 = jnp.transpose(
        edges_matrix_batch.reshape(B, N, N // 128, 128, D_EDGE),
        (0, 1, 2, 4, 3)).reshape(B * N * D_EDGE * N // 16, 16)
    wfeat, wadj, wedge = _sc_gather(gidx, feats_flat, adj16, edges16)
    w_feat = jnp.swapaxes(wfeat.reshape(B, K, N, D_FEAT), 1, 2)
    w_adj = wadj[:, :NPAIR].reshape(B, N, K, K)
    w_edges = wedge[:, :NED].reshape(B, N, K, K, D_EDGE)
    return (w_adj, w_edges, w_feat)
